# Initial kernel scaffold; baseline (speedup 1.0000x reference)
import jax
import jax.numpy as jnp
from jax.experimental import pallas as pl


def kernel(x, gate_w, W1, B1, W2, B2, W3, B3, SW1, SB1, SW2, SB2, SW3, SB3):
    raise NotImplementedError("write your pallas kernel here")


# speedup vs baseline: 2.2625x; 2.2625x over previous
import functools

import jax
import jax.numpy as jnp
from jax.experimental import pallas as pl
from jax.experimental.pallas import tpu as pltpu

DIM = 768
INTER = 768
E = 8
SINTER = 1536
NTOK = 2048


def _mm_nt(a, b):
    return jax.lax.dot_general(
        a, b, (((1,), (1,)), ((), ())), preferred_element_type=jnp.float32)


def _moe_dense_kernel(x_ref, gw_ref, W1_ref, B1_ref, W2_ref, B2_ref,
                      W3_ref, B3_ref, SW1_ref, SB1_ref, SW2_ref, SB2_ref,
                      SW3_ref, SB3_ref, y_ref, gate_ref):
    i = pl.program_id(0)

    @pl.when(i == 0)
    def _gating_and_shared():
        x = x_ref[...]
        scores = _mm_nt(x, gw_ref[...])
        m = jnp.max(scores, axis=-1, keepdims=True)
        p = jnp.exp(scores - m)
        s = p / jnp.sum(p, axis=-1, keepdims=True)
        lane = jax.lax.broadcasted_iota(jnp.int32, (NTOK, E), 1)
        m1 = jnp.max(s, axis=-1, keepdims=True)
        idx1 = jnp.min(jnp.where(s == m1, lane, E), axis=-1, keepdims=True)
        oh1 = lane == idx1
        s_m = jnp.where(oh1, -jnp.inf, s)
        m2 = jnp.max(s_m, axis=-1, keepdims=True)
        idx2 = jnp.min(jnp.where(s_m == m2, lane, E), axis=-1, keepdims=True)
        oh2 = lane == idx2
        gate_ref[...] = jnp.where(jnp.logical_or(oh1, oh2), s, 0.0)
        h = jax.nn.silu(_mm_nt(x, SW1_ref[...]) + SB1_ref[...]) * (
            _mm_nt(x, SW3_ref[...]) + SB3_ref[...])
        y_ref[...] = _mm_nt(h, SW2_ref[...]) + SB2_ref[...]

    @pl.when(i > 0)
    def _expert():
        x = x_ref[...]
        e = i - 1
        lane = jax.lax.broadcasted_iota(jnp.int32, (NTOK, E), 1)
        g = jnp.sum(jnp.where(lane == e, gate_ref[...], 0.0), axis=-1,
                    keepdims=True)
        h = jax.nn.silu(_mm_nt(x, W1_ref[0]) + B1_ref[0]) * (
            _mm_nt(x, W3_ref[0]) + B3_ref[0])
        out_e = _mm_nt(h, W2_ref[0]) + B2_ref[0]
        y_ref[...] += g * out_e


def kernel(x, gate_w, W1, B1, W2, B2, W3, B3, SW1, SB1, SW2, SB2, SW3, SB3):
    ew = lambda i: (jnp.maximum(i - 1, 0), 0, 0)
    whole2 = lambda i: (0, 0)
    grid = (E + 1,)
    return pl.pallas_call(
        _moe_dense_kernel,
        grid=grid,
        in_specs=[
            pl.BlockSpec((NTOK, DIM), whole2),
            pl.BlockSpec((E, DIM), whole2),
            pl.BlockSpec((1, INTER, DIM), ew),
            pl.BlockSpec((1, 1, INTER), ew),
            pl.BlockSpec((1, DIM, INTER), ew),
            pl.BlockSpec((1, 1, DIM), ew),
            pl.BlockSpec((1, INTER, DIM), ew),
            pl.BlockSpec((1, 1, INTER), ew),
            pl.BlockSpec((SINTER, DIM), whole2),
            pl.BlockSpec((1, SINTER), whole2),
            pl.BlockSpec((DIM, SINTER), whole2),
            pl.BlockSpec((1, DIM), whole2),
            pl.BlockSpec((SINTER, DIM), whole2),
            pl.BlockSpec((1, SINTER), whole2),
        ],
        out_specs=pl.BlockSpec((NTOK, DIM), whole2),
        out_shape=jax.ShapeDtypeStruct((NTOK, DIM), jnp.float32),
        scratch_shapes=[pltpu.VMEM((NTOK, E), jnp.float32)],
        compiler_params=pltpu.CompilerParams(
            dimension_semantics=("arbitrary",)),
    )(x, gate_w, W1, B1.reshape(E, 1, INTER), W2, B2.reshape(E, 1, DIM),
      W3, B3.reshape(E, 1, INTER),
      SW1, SB1.reshape(1, SINTER), SW2, SB2.reshape(1, DIM),
      SW3, SB3.reshape(1, SINTER))
